# baseline (speedup 1.0000x reference)
import functools

import jax
import jax.numpy as jnp
from jax import lax
from jax.experimental import pallas as pl
from jax.experimental.pallas import tpu as pltpu
from jax.experimental.pallas import tpu_sc as plsc

L = 16
NW = 32
B = 16384
H = 64
BW = B // NW
CH = 128
NCH = BW // CH
NG = BW // L


def _mf_body(user_hbm, item_hbm, uw_hbm, ub_hbm, iw_hbm, ib_hbm, out_hbm,
             uidx, iidx, uw_v, iw_v, ub_v, ib_v, out_v, sem):
    wid = lax.axis_index("s") * 2 + lax.axis_index("c")
    base = wid * BW

    for j in range(NCH):
        pltpu.sync_copy(user_hbm.at[pl.ds(base + j * CH, CH)], uidx.at[j])
        pltpu.sync_copy(item_hbm.at[pl.ds(base + j * CH, CH)], iidx.at[j])

    copies = []
    for j in range(NCH):
        sl = pl.ds(j * CH, CH)
        copies.append(pltpu.async_copy(uw_hbm.at[uidx.at[j]], uw_v.at[sl], sem))
        copies.append(pltpu.async_copy(iw_hbm.at[iidx.at[j]], iw_v.at[sl], sem))
        copies.append(pltpu.async_copy(ub_hbm.at[uidx.at[j]], ub_v.at[sl], sem))
        copies.append(pltpu.async_copy(ib_hbm.at[iidx.at[j]], ib_v.at[sl], sem))
    for c in copies:
        c.wait()

    lanes = lax.iota(jnp.int32, L)
    shufs = [lanes ^ k for k in (1, 2, 4, 8)]

    dn = lax.GatherDimensionNumbers(
        offset_dims=(), collapsed_slice_dims=(0,), start_index_map=(0,))

    def shuffle(v, idx):
        return lax.gather(v, idx[:, None], dn, slice_sizes=(1,),
                          mode=lax.GatherScatterMode.PROMISE_IN_BOUNDS)

    def hsum(v):
        for sidx in shufs:
            v = v + shuffle(v, sidx)
        return v

    def group(g, carry):
        ub16 = ub_v[pl.ds(g * L, L)]
        ib16 = ib_v[pl.ds(g * L, L)]
        out_acc = jnp.zeros((L,), jnp.float32)
        for r in range(L):
            row = g * L + r
            ubr = ub16[r]
            ibr = ib16[r]
            s = jnp.zeros((L,), jnp.float32)
            for c4 in range(H // L):
                u = uw_v[row, pl.ds(c4 * L, L)] + ubr
                it = iw_v[row, pl.ds(c4 * L, L)] + ibr
                s = s + u * it
            out_acc = jnp.where(lanes == r, hsum(s), out_acc)
        out_v[pl.ds(g * L, L)] = out_acc
        return carry

    lax.fori_loop(0, NG, group, 0)
    pltpu.sync_copy(out_v, out_hbm.at[pl.ds(base, BW)])


@functools.partial(jax.jit, static_argnums=())
def _mf(user, item, user_weight, user_bias, item_weight, item_bias):
    mesh = plsc.VectorSubcoreMesh(core_axis_name="c", subcore_axis_name="s")
    run = pl.kernel(
        _mf_body,
        out_type=jax.ShapeDtypeStruct((B,), jnp.float32),
        mesh=mesh,
        scratch_types=[
            pltpu.VMEM((NCH, CH), jnp.int32),
            pltpu.VMEM((NCH, CH), jnp.int32),
            pltpu.VMEM((BW, H), jnp.float32),
            pltpu.VMEM((BW, H), jnp.float32),
            pltpu.VMEM((BW,), jnp.float32),
            pltpu.VMEM((BW,), jnp.float32),
            pltpu.VMEM((BW,), jnp.float32),
            pltpu.SemaphoreType.DMA,
        ],
        compiler_params=pltpu.CompilerParams(use_tc_tiling_on_sc=False),
    )
    return run(user, item, user_weight, user_bias, item_weight, item_bias)


def kernel(user, item, user_weight, user_bias, item_weight, item_bias, bias):
    out = _mf(user, item, user_weight, user_bias.reshape(-1),
              item_weight, item_bias.reshape(-1))
    return out + bias
